# baseline (speedup 1.0000x reference)
import functools

import jax
import jax.numpy as jnp
from jax import lax
from jax.experimental import pallas as pl
from jax.experimental.pallas import tpu as pltpu
from jax.experimental.pallas import tpu_sc as plsc

LANES = 16


def _build_gather(n_pair, emb_dim, n_workers, pairs_per_chunk):
    pairs_per_w = n_pair // n_workers
    n_chunks = pairs_per_w // pairs_per_chunk
    row2 = 2 * emb_dim
    p = pairs_per_chunk

    mesh = plsc.VectorSubcoreMesh(core_axis_name="c", subcore_axis_name="s")

    @functools.partial(
        pl.kernel,
        mesh=mesh,
        compiler_params=pltpu.CompilerParams(
            needs_layout_passes=False, use_tc_tiling_on_sc=False),
        out_type=jax.ShapeDtypeStruct((n_pair, row2), jnp.float32),
        scratch_types=[
            pltpu.VMEM((2 * p,), jnp.int32),
            pltpu.VMEM((2 * p, emb_dim), jnp.float32),
            pltpu.VMEM((p, row2), jnp.float32),
            pltpu.SemaphoreType.DMA,
            pltpu.SemaphoreType.DMA,
        ],
    )
    def gather_kernel(idx_hbm, w_hbm, out_hbm, idx_v, rows_v, out_v, gsem, osem):
        n_cores = mesh.num_cores
        wid = lax.axis_index("s") * n_cores + lax.axis_index("c")
        base_pair = wid * pairs_per_w

        even = 2 * lax.iota(jnp.int32, 16)
        odd = even + 1

        def chunk_body(c, carry):
            pair0 = base_pair + c * p
            pltpu.sync_copy(idx_hbm.at[pl.ds(pair0 * 2, 2 * p)], idx_v)
            pltpu.async_copy(w_hbm.at[idx_v], rows_v, gsem).wait()

            def pair_body(j, carry2):
                vj = jnp.full((LANES,), j, jnp.int32)
                for g in range(emb_dim // LANES):
                    a = rows_v[2 * j, pl.ds(g * LANES, LANES)]
                    b = rows_v[2 * j + 1, pl.ds(g * LANES, LANES)]
                    plsc.store_scatter(out_v, [vj, g * 2 * LANES + even], a)
                    plsc.store_scatter(out_v, [vj, g * 2 * LANES + odd], b)
                return carry2

            lax.fori_loop(0, p, pair_body, 0, unroll=4)
            pltpu.async_copy(out_v, out_hbm.at[pl.ds(pair0, p)], osem).wait()
            return carry

        lax.fori_loop(0, n_chunks, chunk_body, 0)

    return gather_kernel


def kernel(avnn_tensor, weight):
    bsz, seq, two = avnn_tensor.shape
    assert two == 2
    emb_dim = weight.shape[1]
    n_pair = bsz * seq

    idx_flat = avnn_tensor.reshape(-1).astype(jnp.int32)
    n_workers = 32
    gather = _build_gather(n_pair, emb_dim, n_workers, pairs_per_chunk=64)
    out_flat = gather(idx_flat, weight)
    return out_flat.reshape(bsz, seq, emb_dim, 2)


# speedup vs baseline: 1.1806x; 1.1806x over previous
import functools

import jax
import jax.numpy as jnp
from jax import lax
from jax.experimental import pallas as pl
from jax.experimental.pallas import tpu as pltpu
from jax.experimental.pallas import tpu_sc as plsc

LANES = 16
NBUF = 2


def _build_gather(n_pair, emb_dim, n_workers, pairs_per_chunk):
    pairs_per_w = n_pair // n_workers
    n_chunks = pairs_per_w // pairs_per_chunk
    assert n_chunks % NBUF == 0
    row2 = 2 * emb_dim
    p = pairs_per_chunk
    ipc = 2 * p

    mesh = plsc.VectorSubcoreMesh(core_axis_name="c", subcore_axis_name="s")

    @functools.partial(
        pl.kernel,
        mesh=mesh,
        compiler_params=pltpu.CompilerParams(
            needs_layout_passes=False, use_tc_tiling_on_sc=False),
        out_type=jax.ShapeDtypeStruct((n_pair, row2), jnp.float32),
        scratch_types=[
            pltpu.VMEM((n_chunks, ipc), jnp.int32),
            pltpu.VMEM((NBUF, ipc, emb_dim), jnp.float32),
            pltpu.VMEM((NBUF, p, row2), jnp.float32),
            pltpu.SemaphoreType.DMA,
            pltpu.SemaphoreType.DMA,
            pltpu.SemaphoreType.DMA,
            pltpu.SemaphoreType.DMA,
        ],
    )
    def gather_kernel(idx_hbm, w_hbm, out_hbm, idx_v, rows_v, out_v,
                      gsem0, gsem1, osem0, osem1):
        n_cores = mesh.num_cores
        wid = lax.axis_index("s") * n_cores + lax.axis_index("c")
        base_chunk = wid * n_chunks
        base_pair = wid * pairs_per_w

        gsems = (gsem0, gsem1)
        osems = (osem0, osem1)

        even = 2 * lax.iota(jnp.int32, 16)
        odd = even + 1

        pltpu.sync_copy(idx_hbm.at[pl.ds(base_chunk, n_chunks)], idx_v)

        def start_gather(c, b):
            pltpu.async_copy(w_hbm.at[idx_v.at[c]], rows_v.at[b], gsems[b])

        def wait_gather(c, b):
            pltpu.make_async_copy(
                w_hbm.at[idx_v.at[c]], rows_v.at[b], gsems[b]).wait()

        def start_out(c, b):
            pltpu.async_copy(
                out_v.at[b], out_hbm.at[pl.ds(base_pair + c * p, p)],
                osems[b])

        def wait_out(c, b):
            pltpu.make_async_copy(
                out_v.at[b], out_hbm.at[pl.ds(base_pair + c * p, p)],
                osems[b]).wait()

        for b in range(NBUF):
            start_gather(b, b)

        def outer(g, carry):
            for b in range(NBUF):
                c = g * NBUF + b
                wait_gather(c, b)

                @pl.when(g > 0)
                def _():
                    wait_out(c - NBUF, b)

                def pair_body(j, carry2):
                    vj = jnp.full((LANES,), j, jnp.int32)
                    for grp in range(emb_dim // LANES):
                        a = rows_v[b, 2 * j, pl.ds(grp * LANES, LANES)]
                        bb = rows_v[b, 2 * j + 1, pl.ds(grp * LANES, LANES)]
                        plsc.store_scatter(
                            out_v.at[b], [vj, grp * 2 * LANES + even], a)
                        plsc.store_scatter(
                            out_v.at[b], [vj, grp * 2 * LANES + odd], bb)
                    return carry2

                lax.fori_loop(0, p, pair_body, 0, unroll=4)
                start_out(c, b)

                @pl.when(c + NBUF < n_chunks)
                def _():
                    start_gather(c + NBUF, b)
            return carry

        lax.fori_loop(0, n_chunks // NBUF, outer, 0)

        for b in range(NBUF):
            wait_out(n_chunks - NBUF + b, b)

    return gather_kernel


def kernel(avnn_tensor, weight):
    bsz, seq, two = avnn_tensor.shape
    assert two == 2
    emb_dim = weight.shape[1]
    n_pair = bsz * seq
    n_workers = 32
    p = 64

    idx_flat = avnn_tensor.reshape(n_pair * 2 // (2 * p), 2 * p)
    idx_flat = idx_flat.astype(jnp.int32)
    gather = _build_gather(n_pair, emb_dim, n_workers, pairs_per_chunk=p)
    out_flat = gather(idx_flat, weight)
    return out_flat.reshape(bsz, seq, emb_dim, 2)


# speedup vs baseline: 1.7411x; 1.4748x over previous
import functools

import jax
import jax.numpy as jnp
from jax import lax
from jax.experimental import pallas as pl
from jax.experimental.pallas import tpu as pltpu
from jax.experimental.pallas import tpu_sc as plsc

LANES = 16
NBUF = 2


def _build_gather(bsz, seq, emb_dim, n_workers):
    bt = bsz // n_workers
    rpc = 2 * bt
    assert bt % 8 == 0 and bt <= 128

    mesh = plsc.VectorSubcoreMesh(core_axis_name="c", subcore_axis_name="s")

    @functools.partial(
        pl.kernel,
        mesh=mesh,
        compiler_params=pltpu.CompilerParams(
            needs_layout_passes=False, use_tc_tiling_on_sc=False),
        out_type=jax.ShapeDtypeStruct((seq, emb_dim, n_workers, rpc),
                                      jnp.float32),
        scratch_types=[
            pltpu.VMEM((seq, rpc), jnp.int32),
            pltpu.VMEM((NBUF, rpc, emb_dim), jnp.float32),
            pltpu.VMEM((NBUF, emb_dim, rpc), jnp.float32),
            pltpu.SemaphoreType.DMA,
            pltpu.SemaphoreType.DMA,
            pltpu.SemaphoreType.DMA,
            pltpu.SemaphoreType.DMA,
        ],
    )
    def gather_kernel(idx_hbm, w_hbm, out_hbm, idx_v, rows_v, out_v,
                      gsem0, gsem1, osem0, osem1):
        n_cores = mesh.num_cores
        wid = lax.axis_index("s") * n_cores + lax.axis_index("c")

        gsems = (gsem0, gsem1)
        osems = (osem0, osem1)

        lane = lax.iota(jnp.int32, 16)

        pltpu.sync_copy(idx_hbm.at[:, wid], idx_v)

        def start_gather(c, b):
            for h in range(2):
                pltpu.async_copy(
                    w_hbm.at[idx_v.at[c, pl.ds(h * bt, bt)]],
                    rows_v.at[b, pl.ds(h * bt, bt)], gsems[b])

        def wait_gather(c, b):
            for h in range(2):
                pltpu.make_async_copy(
                    w_hbm.at[idx_v.at[c, pl.ds(h * bt, bt)]],
                    rows_v.at[b, pl.ds(h * bt, bt)], gsems[b]).wait()

        def start_out(c, b):
            pltpu.async_copy(out_v.at[b], out_hbm.at[c, :, wid], osems[b])

        def wait_out(c, b):
            pltpu.make_async_copy(
                out_v.at[b], out_hbm.at[c, :, wid], osems[b]).wait()

        for b in range(NBUF):
            start_gather(b, b)

        def outer(g, carry):
            for b in range(NBUF):
                c = g * NBUF + b
                wait_gather(c, b)

                @pl.when(g > 0)
                def _():
                    wait_out(c - NBUF, b)

                def row_body(r, carry2):
                    vr = jnp.full((LANES,), r, jnp.int32)
                    for grp in range(emb_dim // LANES):
                        a = rows_v[b, r, pl.ds(grp * LANES, LANES)]
                        plsc.store_scatter(
                            out_v.at[b], [grp * LANES + lane, vr], a)
                    return carry2

                lax.fori_loop(0, rpc, row_body, 0, unroll=4)
                start_out(c, b)

                @pl.when(c + NBUF < seq)
                def _():
                    start_gather(c + NBUF, b)
            return carry

        lax.fori_loop(0, seq // NBUF, outer, 0)

        for b in range(NBUF):
            wait_out(seq - NBUF + b, b)

    return gather_kernel


def kernel(avnn_tensor, weight):
    bsz, seq, two = avnn_tensor.shape
    assert two == 2
    emb_dim = weight.shape[1]
    n_workers = 32
    bt = bsz // n_workers

    idx = avnn_tensor.astype(jnp.int32)
    idx = idx.transpose(1, 0, 2).reshape(seq, n_workers, bt, 2)
    idx = idx.transpose(0, 1, 3, 2).reshape(seq, n_workers, 2 * bt)

    gather = _build_gather(bsz, seq, emb_dim, n_workers)
    out4 = gather(idx, weight)

    out = out4.reshape(seq, emb_dim, n_workers, 2, bt)
    out = out.transpose(2, 4, 0, 1, 3).reshape(bsz, seq, emb_dim, 2)
    return out


# speedup vs baseline: 3.2340x; 1.8575x over previous
import functools

import jax
import jax.numpy as jnp
from jax import lax
from jax.experimental import pallas as pl
from jax.experimental.pallas import tpu as pltpu
from jax.experimental.pallas import tpu_sc as plsc

LANES = 16
NBUF = 2


def _build_gather(bsz, seq, emb_dim, n_workers):
    bt = bsz // n_workers
    rpc = 2 * bt
    assert bt % 8 == 0 and bt <= 128

    mesh = plsc.VectorSubcoreMesh(core_axis_name="c", subcore_axis_name="s")

    @functools.partial(
        pl.kernel,
        mesh=mesh,
        compiler_params=pltpu.CompilerParams(
            needs_layout_passes=False, use_tc_tiling_on_sc=False),
        out_type=jax.ShapeDtypeStruct((seq, emb_dim, n_workers, rpc),
                                      jnp.float32),
        scratch_types=[
            pltpu.VMEM((seq, rpc), jnp.int32),
            pltpu.VMEM((NBUF, rpc, emb_dim), jnp.float32),
            pltpu.VMEM((NBUF, emb_dim, rpc + 1), jnp.float32),
            pltpu.SemaphoreType.DMA,
            pltpu.SemaphoreType.DMA,
            pltpu.SemaphoreType.DMA,
            pltpu.SemaphoreType.DMA,
        ],
    )
    def gather_kernel(idx_hbm, w_hbm, out_hbm, idx_v, rows_v, out_v,
                      gsem0, gsem1, osem0, osem1):
        n_cores = mesh.num_cores
        wid = lax.axis_index("s") * n_cores + lax.axis_index("c")

        gsems = (gsem0, gsem1)
        osems = (osem0, osem1)

        lane = lax.iota(jnp.int32, 16)

        pltpu.sync_copy(idx_hbm.at[:, wid], idx_v)

        def start_gather(c, b):
            for h in range(2):
                pltpu.async_copy(
                    w_hbm.at[idx_v.at[c, pl.ds(h * bt, bt)]],
                    rows_v.at[b, pl.ds(h * bt, bt)], gsems[b])

        def wait_gather(c, b):
            for h in range(2):
                pltpu.make_async_copy(
                    w_hbm.at[idx_v.at[c, pl.ds(h * bt, bt)]],
                    rows_v.at[b, pl.ds(h * bt, bt)], gsems[b]).wait()

        def start_out(c, b):
            pltpu.async_copy(out_v.at[b, :, pl.ds(0, rpc)],
                             out_hbm.at[c, :, wid], osems[b])

        def wait_out(c, b):
            pltpu.make_async_copy(
                out_v.at[b, :, pl.ds(0, rpc)],
                out_hbm.at[c, :, wid], osems[b]).wait()

        for b in range(NBUF):
            start_gather(b, b)

        def outer(g, carry):
            for b in range(NBUF):
                c = g * NBUF + b
                wait_gather(c, b)

                @pl.when(g > 0)
                def _():
                    wait_out(c - NBUF, b)

                def row_body(r, carry2):
                    vr = jnp.full((LANES,), r, jnp.int32)
                    for grp in range(emb_dim // LANES):
                        a = rows_v[b, r, pl.ds(grp * LANES, LANES)]
                        plsc.store_scatter(
                            out_v.at[b], [grp * LANES + lane, vr], a)
                    return carry2

                lax.fori_loop(0, rpc, row_body, 0, unroll=4)
                start_out(c, b)

                @pl.when(c + NBUF < seq)
                def _():
                    start_gather(c + NBUF, b)
            return carry

        lax.fori_loop(0, seq // NBUF, outer, 0)

        for b in range(NBUF):
            wait_out(seq - NBUF + b, b)

    return gather_kernel


def kernel(avnn_tensor, weight):
    bsz, seq, two = avnn_tensor.shape
    assert two == 2
    emb_dim = weight.shape[1]
    n_workers = 32
    bt = bsz // n_workers

    idx = avnn_tensor.astype(jnp.int32)
    idx = idx.transpose(1, 0, 2).reshape(seq, n_workers, bt, 2)
    idx = idx.transpose(0, 1, 3, 2).reshape(seq, n_workers, 2 * bt)

    gather = _build_gather(bsz, seq, emb_dim, n_workers)
    out4 = gather(idx, weight)

    out = out4.reshape(seq, emb_dim, n_workers, 2, bt)
    out = out.transpose(2, 4, 0, 1, 3).reshape(bsz, seq, emb_dim, 2)
    return out


# speedup vs baseline: 3.8239x; 1.1824x over previous
import functools

import jax
import jax.numpy as jnp
from jax import lax
from jax.experimental import pallas as pl
from jax.experimental.pallas import tpu as pltpu
from jax.experimental.pallas import tpu_sc as plsc

LANES = 16
NBUF = 2


def _build_gather(bsz, seq, emb_dim, n_workers):
    bt = bsz // n_workers
    rpc = 2 * bt
    assert bt % 8 == 0 and bt <= 128

    mesh = plsc.VectorSubcoreMesh(core_axis_name="c", subcore_axis_name="s")

    @functools.partial(
        pl.kernel,
        mesh=mesh,
        compiler_params=pltpu.CompilerParams(
            needs_layout_passes=False, use_tc_tiling_on_sc=False),
        out_type=jax.ShapeDtypeStruct((seq, emb_dim, n_workers, rpc),
                                      jnp.float32),
        scratch_types=[
            pltpu.VMEM((seq, rpc), jnp.int32),
            pltpu.VMEM((NBUF, rpc, emb_dim), jnp.float32),
            pltpu.VMEM((NBUF, emb_dim, rpc + 1), jnp.float32),
            pltpu.SemaphoreType.DMA,
            pltpu.SemaphoreType.DMA,
            pltpu.SemaphoreType.DMA,
            pltpu.SemaphoreType.DMA,
        ],
    )
    def gather_kernel(idx_hbm, w_hbm, out_hbm, idx_v, rows_v, out_v,
                      gsem0, gsem1, osem0, osem1):
        n_cores = mesh.num_cores
        wid = lax.axis_index("s") * n_cores + lax.axis_index("c")

        gsems = (gsem0, gsem1)
        osems = (osem0, osem1)

        lane = lax.iota(jnp.int32, 16)

        pltpu.sync_copy(idx_hbm.at[:, wid], idx_v)

        def start_gather(c, b):
            for h in range(2):
                pltpu.async_copy(
                    w_hbm.at[idx_v.at[c, pl.ds(h * bt, bt)]],
                    rows_v.at[b, pl.ds(h * bt, bt)], gsems[b])

        def wait_gather(c, b):
            for h in range(2):
                pltpu.make_async_copy(
                    w_hbm.at[idx_v.at[c, pl.ds(h * bt, bt)]],
                    rows_v.at[b, pl.ds(h * bt, bt)], gsems[b]).wait()

        def start_out(c, b):
            pltpu.async_copy(out_v.at[b, :, pl.ds(0, rpc)],
                             out_hbm.at[c, :, wid], osems[b])

        def wait_out(c, b):
            pltpu.make_async_copy(
                out_v.at[b, :, pl.ds(0, rpc)],
                out_hbm.at[c, :, wid], osems[b]).wait()

        for b in range(NBUF):
            start_gather(b, b)

        def outer(g, carry):
            for b in range(NBUF):
                c = g * NBUF + b
                wait_gather(c, b)

                @pl.when(g > 0)
                def _():
                    wait_out(c - NBUF, b)

                def row_body(r, carry2):
                    vr = jnp.full((LANES,), r, jnp.int32)
                    vals = [rows_v[b, r, pl.ds(grp * LANES, LANES)]
                            for grp in range(emb_dim // LANES)]
                    for grp, a in enumerate(vals):
                        plsc.store_scatter(
                            out_v.at[b], [grp * LANES + lane, vr], a)
                    return carry2

                lax.fori_loop(0, rpc, row_body, 0, unroll=4)
                start_out(c, b)

                @pl.when(c + NBUF < seq)
                def _():
                    start_gather(c + NBUF, b)
            return carry

        lax.fori_loop(0, seq // NBUF, outer, 0)

        for b in range(NBUF):
            wait_out(seq - NBUF + b, b)

    return gather_kernel


def kernel(avnn_tensor, weight):
    bsz, seq, two = avnn_tensor.shape
    assert two == 2
    emb_dim = weight.shape[1]
    n_workers = 32
    bt = bsz // n_workers

    idx = avnn_tensor.astype(jnp.int32)
    idx = idx.transpose(1, 0, 2).reshape(seq, n_workers, bt, 2)
    idx = idx.transpose(0, 1, 3, 2).reshape(seq, n_workers, 2 * bt)

    gather = _build_gather(bsz, seq, emb_dim, n_workers)
    out4 = gather(idx, weight)

    out = out4.reshape(seq, emb_dim, n_workers, 2, bt)
    out = out.transpose(2, 4, 0, 1, 3).reshape(bsz, seq, emb_dim, 2)
    return out


# speedup vs baseline: 3.8338x; 1.0026x over previous
import functools

import jax
import jax.numpy as jnp
from jax import lax
from jax.experimental import pallas as pl
from jax.experimental.pallas import tpu as pltpu
from jax.experimental.pallas import tpu_sc as plsc

LANES = 16
NBUF = 2


def _build_gather(bsz, seq, emb_dim, n_workers):
    bt = bsz // n_workers
    rpc = 2 * bt
    assert bt % 8 == 0 and bt <= 128

    mesh = plsc.VectorSubcoreMesh(core_axis_name="c", subcore_axis_name="s")

    @functools.partial(
        pl.kernel,
        mesh=mesh,
        compiler_params=pltpu.CompilerParams(
            needs_layout_passes=False, use_tc_tiling_on_sc=False),
        out_type=jax.ShapeDtypeStruct((seq, emb_dim, n_workers, rpc),
                                      jnp.float32),
        scratch_types=[
            pltpu.VMEM((seq, rpc), jnp.int32),
            pltpu.VMEM((NBUF, rpc, emb_dim), jnp.float32),
            pltpu.VMEM((NBUF, emb_dim, rpc + 1), jnp.float32),
            pltpu.SemaphoreType.DMA,
            pltpu.SemaphoreType.DMA,
            pltpu.SemaphoreType.DMA,
            pltpu.SemaphoreType.DMA,
        ],
    )
    def gather_kernel(idx_hbm, w_hbm, out_hbm, idx_v, rows_v, out_v,
                      gsem0, gsem1, osem0, osem1):
        n_cores = mesh.num_cores
        wid = lax.axis_index("s") * n_cores + lax.axis_index("c")

        gsems = (gsem0, gsem1)
        osems = (osem0, osem1)

        lane = lax.iota(jnp.int32, 16)

        pltpu.sync_copy(idx_hbm.at[:, wid], idx_v)

        def start_gather(c, b):
            for h in range(2):
                pltpu.async_copy(
                    w_hbm.at[idx_v.at[c, pl.ds(h * bt, bt)]],
                    rows_v.at[b, pl.ds(h * bt, bt)], gsems[b])

        def wait_gather(c, b):
            for h in range(2):
                pltpu.make_async_copy(
                    w_hbm.at[idx_v.at[c, pl.ds(h * bt, bt)]],
                    rows_v.at[b, pl.ds(h * bt, bt)], gsems[b]).wait()

        def start_out(c, b):
            pltpu.async_copy(out_v.at[b, :, pl.ds(0, rpc)],
                             out_hbm.at[c, :, wid], osems[b])

        def wait_out(c, b):
            pltpu.make_async_copy(
                out_v.at[b, :, pl.ds(0, rpc)],
                out_hbm.at[c, :, wid], osems[b]).wait()

        for b in range(NBUF):
            start_gather(b, b)

        def outer(g, carry):
            for b in range(NBUF):
                c = g * NBUF + b
                wait_gather(c, b)

                @pl.when(g > 0)
                def _():
                    wait_out(c - NBUF, b)

                def row_body(r, carry2):
                    vr = jnp.full((LANES,), r, jnp.int32)
                    vals = [rows_v[b, r, pl.ds(grp * LANES, LANES)]
                            for grp in range(emb_dim // LANES)]
                    for grp, a in enumerate(vals):
                        plsc.store_scatter(
                            out_v.at[b], [grp * LANES + lane, vr], a)
                    return carry2

                lax.fori_loop(0, rpc, row_body, 0, unroll=8)
                start_out(c, b)

                @pl.when(c + NBUF < seq)
                def _():
                    start_gather(c + NBUF, b)
            return carry

        lax.fori_loop(0, seq // NBUF, outer, 0)

        for b in range(NBUF):
            wait_out(seq - NBUF + b, b)

    return gather_kernel


def kernel(avnn_tensor, weight):
    bsz, seq, two = avnn_tensor.shape
    assert two == 2
    emb_dim = weight.shape[1]
    n_workers = 32
    bt = bsz // n_workers

    idx = avnn_tensor.astype(jnp.int32)
    idx = idx.transpose(1, 0, 2).reshape(seq, n_workers, bt, 2)
    idx = idx.transpose(0, 1, 3, 2).reshape(seq, n_workers, 2 * bt)

    gather = _build_gather(bsz, seq, emb_dim, n_workers)
    out4 = gather(idx, weight)

    out = out4.reshape(seq, emb_dim, n_workers, 2, bt)
    out = out.transpose(2, 4, 0, 1, 3).reshape(bsz, seq, emb_dim, 2)
    return out
